# baseline (speedup 1.0000x reference)
import jax
import jax.numpy as jnp
from jax.experimental import pallas as pl
from jax.experimental.pallas import tpu as pltpu

N = 10000
DIN = 256
DOUT = 256

BM = 400
HALF = BM // 2


def _body(adj_top_ref, adj_bot_ref, x_ref, w_ref, out_ref):
    gt = jnp.dot(adj_top_ref[...], x_ref[...], preferred_element_type=jnp.float32)
    gb = jnp.dot(adj_bot_ref[...], x_ref[...], preferred_element_type=jnp.float32)
    out_ref[:HALF, :] = jax.lax.dot_general(
        gt, w_ref[...],
        dimension_numbers=(((1,), (1,)), ((), ())),
        preferred_element_type=jnp.float32,
    )
    out_ref[HALF:, :] = jax.lax.dot_general(
        gb, w_ref[...],
        dimension_numbers=(((1,), (1,)), ((), ())),
        preferred_element_type=jnp.float32,
    )


@jax.jit
def kernel(x, adj, W):
    return pl.pallas_call(
        _body,
        grid=(N // BM,),
        in_specs=[
            pl.BlockSpec((HALF, N), lambda i: (2 * i, 0)),
            pl.BlockSpec((HALF, N), lambda i: (2 * i + 1, 0)),
            pl.BlockSpec((N, DIN), lambda i: (0, 0)),
            pl.BlockSpec((DOUT, DIN), lambda i: (0, 0)),
        ],
        out_specs=pl.BlockSpec((BM, DOUT), lambda i: (i, 0)),
        out_shape=jax.ShapeDtypeStruct((N, DOUT), jnp.float32),
        compiler_params=pltpu.CompilerParams(
            dimension_semantics=("parallel",),
        ),
    )(adj, adj, x, W)
